# baseline (speedup 1.0000x reference)
import jax
import jax.numpy as jnp
from jax import lax
from jax.experimental import pallas as pl
from jax.experimental.pallas import tpu as pltpu

_V = 100000
_K = 64
_LAMB = 0.01
_B = 256
_TV = 2000
_NT = _V // _TV
_S = 4
_CH = _TV // _S


def _fused_kernel(q_ref, emb_ref, acc_ref, ebuf, sems):
    i = pl.program_id(0)

    def dma(step, slot, j):
        return pltpu.make_async_copy(
            emb_ref.at[pl.ds(step * _TV + j * _CH, _CH)],
            ebuf.at[slot, pl.ds(j * _CH, _CH)],
            sems.at[slot, j],
        )

    @pl.when(i == 0)
    def _prologue():
        for j in range(_S):
            dma(0, 0, j).start()
        acc_ref[...] = jnp.zeros_like(acc_ref)

    slot = lax.rem(i, 2)
    nxt = lax.rem(i + 1, 2)

    @pl.when(i + 1 < _NT)
    def _prefetch():
        for j in range(_S):
            dma(i + 1, nxt, j).start()

    for j in range(_S):
        dma(i, slot, j).wait()

    e = jnp.concatenate(
        [ebuf[slot, :, 0, :], ebuf[slot, :, 1, :], ebuf[slot, :, 2, :]],
        axis=1)
    logits = jax.lax.dot_general(
        q_ref[...], e.astype(jnp.bfloat16),
        (((1,), (1,)), ((), ())), preferred_element_type=jnp.float32)
    acc_ref[...] += jnp.sum(jnp.exp(logits), axis=1, keepdims=True)


def kernel(triples, emb):
    sub = triples[:, 0]
    pred = triples[:, 1]
    obj = triples[:, 2]

    s = jnp.take(emb, sub, axis=0)
    p = jnp.take(emb, pred, axis=0)
    o = jnp.take(emb, obj, axis=0)

    q_o = jnp.concatenate([s[:, 2] * p[:, 2], s[:, 1] * p[:, 1], s[:, 0] * p[:, 0]], axis=-1)
    q_s = jnp.concatenate([p[:, 0] * o[:, 2], p[:, 1] * o[:, 1], p[:, 2] * o[:, 0]], axis=-1)
    q = jnp.concatenate([q_o, q_s], axis=0).astype(jnp.bfloat16)

    acc = pl.pallas_call(
        _fused_kernel,
        grid=(_NT,),
        in_specs=[
            pl.BlockSpec((2 * _B, 3 * _K), lambda i: (0, 0)),
            pl.BlockSpec(memory_space=pltpu.MemorySpace.HBM),
        ],
        out_specs=pl.BlockSpec((2 * _B, 1), lambda i: (0, 0)),
        out_shape=jax.ShapeDtypeStruct((2 * _B, 1), jnp.float32),
        scratch_shapes=[
            pltpu.VMEM((2, _TV, 3, _K), jnp.float32),
            pltpu.SemaphoreType.DMA((2, _S)),
        ],
    )(q, emb)

    score = jnp.sum(s[:, 0] * p[:, 0] * o[:, 2]
                    + s[:, 1] * p[:, 1] * o[:, 1]
                    + s[:, 2] * p[:, 2] * o[:, 0], axis=-1)
    es = jnp.exp(score)
    lse_o = jnp.log(acc[:_B, 0] - es)
    lse_s = jnp.log(acc[_B:, 0] - es)
    reg = (_LAMB / 3.0) * jnp.sum(jnp.abs(s) ** 3 + jnp.abs(p) ** 3 + jnp.abs(o) ** 3,
                                  axis=(1, 2))
    total_loss = jnp.sum(-2.0 * score + lse_o + lse_s + reg)
    return score, total_loss


# speedup vs baseline: 1.4586x; 1.4586x over previous
import jax
import jax.numpy as jnp
from jax.experimental import pallas as pl
from jax.experimental.pallas import tpu as pltpu

_V = 100000
_K = 64
_LAMB = 0.01
_B = 256
_TV = 4000
_NT = _V // _TV


def _fused_kernel(q_ref, e_ref, acc_ref):
    i = pl.program_id(0)

    @pl.when(i == 0)
    def _init():
        acc_ref[...] = jnp.zeros_like(acc_ref)

    logits = jax.lax.dot_general(
        q_ref[...], e_ref[...],
        (((1,), (1,)), ((), ())), preferred_element_type=jnp.float32)
    acc_ref[...] += jnp.sum(jnp.exp(logits), axis=1, keepdims=True)


def kernel(triples, emb):
    sub = triples[:, 0]
    pred = triples[:, 1]
    obj = triples[:, 2]

    s = jnp.take(emb, sub, axis=0)
    p = jnp.take(emb, pred, axis=0)
    o = jnp.take(emb, obj, axis=0)

    q_o = jnp.concatenate([s[:, 2] * p[:, 2], s[:, 1] * p[:, 1], s[:, 0] * p[:, 0]], axis=-1)
    q_s = jnp.concatenate([p[:, 0] * o[:, 2], p[:, 1] * o[:, 1], p[:, 2] * o[:, 0]], axis=-1)
    q = jnp.concatenate([q_o, q_s], axis=0).astype(jnp.bfloat16)

    e = emb.reshape(_V, 3 * _K).astype(jnp.bfloat16)

    acc = pl.pallas_call(
        _fused_kernel,
        grid=(_NT,),
        in_specs=[
            pl.BlockSpec((2 * _B, 3 * _K), lambda i: (0, 0)),
            pl.BlockSpec((_TV, 3 * _K), lambda i: (i, 0)),
        ],
        out_specs=pl.BlockSpec((2 * _B, 1), lambda i: (0, 0)),
        out_shape=jax.ShapeDtypeStruct((2 * _B, 1), jnp.float32),
    )(q, e)

    score = jnp.sum(s[:, 0] * p[:, 0] * o[:, 2]
                    + s[:, 1] * p[:, 1] * o[:, 1]
                    + s[:, 2] * p[:, 2] * o[:, 0], axis=-1)
    es = jnp.exp(score)
    lse_o = jnp.log(acc[:_B, 0] - es)
    lse_s = jnp.log(acc[_B:, 0] - es)
    reg = (_LAMB / 3.0) * jnp.sum(jnp.abs(s) ** 3 + jnp.abs(p) ** 3 + jnp.abs(o) ** 3,
                                  axis=(1, 2))
    total_loss = jnp.sum(-2.0 * score + lse_o + lse_s + reg)
    return score, total_loss
